# baseline (speedup 1.0000x reference)
import functools

import jax
import jax.numpy as jnp
from jax import lax
from jax.experimental import pallas as pl
from jax.experimental.pallas import tpu as pltpu
from jax.experimental.pallas import tpu_sc as plsc

BATCH = 16384
HIDDEN = 64
GAMMA = 12.0

NUM_CORES = 2
NUM_SUBCORES = 16
NW = NUM_CORES * NUM_SUBCORES
B_PER_W = BATCH // NW
CHUNK = 128
N_CHUNKS = B_PER_W // CHUNK
LANES = 16
UNROLL = 8


def _sc_body(heads_hbm, rels_hbm, tails_hbm, ent_hbm, rel_hbm, out_hbm,
             idx_h, idx_r, idx_t, rows_h, rows_r, rows_t, out_v, sem):
    wid = lax.axis_index("s") * NUM_CORES + lax.axis_index("c")
    crow = wid * N_CHUNKS

    pltpu.sync_copy(heads_hbm.at[pl.ds(crow, N_CHUNKS)], idx_h)
    pltpu.sync_copy(rels_hbm.at[pl.ds(crow, N_CHUNKS)], idx_r)
    pltpu.sync_copy(tails_hbm.at[pl.ds(crow, N_CHUNKS)], idx_t)

    copies = []
    for j in range(N_CHUNKS):
        dst = pl.ds(j * CHUNK, CHUNK)
        copies.append(pltpu.async_copy(ent_hbm.at[idx_h.at[j]], rows_h.at[dst], sem))
        copies.append(pltpu.async_copy(rel_hbm.at[idx_r.at[j]], rows_r.at[dst], sem))
        copies.append(pltpu.async_copy(ent_hbm.at[idx_t.at[j]], rows_t.at[dst], sem))
    for c in copies:
        c.wait()

    lane = lax.iota(jnp.int32, LANES)
    dnums = lax.GatherDimensionNumbers(
        offset_dims=(), collapsed_slice_dims=(0,), start_index_map=(0,))

    def _shuffle(x, idx):
        return lax.gather(x, idx[:, None], dnums, slice_sizes=(1,),
                          mode=lax.GatherScatterMode.PROMISE_IN_BOUNDS)

    def row_group(g, carry):
        out_vec = jnp.zeros((LANES,), jnp.float32)
        for u in range(LANES):
            i = g * LANES + u
            acc = None
            for k in range(HIDDEN // LANES):
                sl = pl.ds(k * LANES, LANES)
                d = jnp.abs(rows_h[i, sl] + rows_r[i, sl] - rows_t[i, sl])
                acc = d if acc is None else acc + d
            for sh in (8, 4, 2, 1):
                acc = acc + _shuffle(acc, lane ^ sh)
            out_vec = jnp.where(lane == u, GAMMA - acc, out_vec)
        out_v[pl.ds(g * LANES, LANES)] = out_vec
        return carry

    lax.fori_loop(0, B_PER_W // LANES, row_group, 0, unroll=False)

    pltpu.sync_copy(out_v, out_hbm.at[pl.ds(wid * B_PER_W, B_PER_W)])


@functools.partial(jax.jit, static_argnames=())
def _score(heads, rels, tails, entity_embedding, relation_embedding):
    mesh = plsc.VectorSubcoreMesh(
        core_axis_name="c", subcore_axis_name="s",
        num_cores=NUM_CORES, num_subcores=NUM_SUBCORES)
    fn = functools.partial(
        pl.kernel,
        out_type=jax.ShapeDtypeStruct((BATCH,), jnp.float32),
        mesh=mesh,
        scratch_types=[
            pltpu.VMEM((N_CHUNKS, CHUNK), jnp.int32),
            pltpu.VMEM((N_CHUNKS, CHUNK), jnp.int32),
            pltpu.VMEM((N_CHUNKS, CHUNK), jnp.int32),
            pltpu.VMEM((B_PER_W, HIDDEN), jnp.float32),
            pltpu.VMEM((B_PER_W, HIDDEN), jnp.float32),
            pltpu.VMEM((B_PER_W, HIDDEN), jnp.float32),
            pltpu.VMEM((B_PER_W,), jnp.float32),
            pltpu.SemaphoreType.DMA,
        ],
        compiler_params=pltpu.CompilerParams(use_tc_tiling_on_sc=False),
    )(_sc_body)
    return fn(heads, rels, tails, entity_embedding, relation_embedding)


def kernel(sample, entity_embedding, relation_embedding):
    sample = sample.astype(jnp.int32)
    heads = sample[:, 0].reshape(BATCH // CHUNK, CHUNK)
    rels = sample[:, 1].reshape(BATCH // CHUNK, CHUNK)
    tails = sample[:, 2].reshape(BATCH // CHUNK, CHUNK)
    score = _score(heads, rels, tails, entity_embedding, relation_embedding)
    return score.reshape(BATCH, 1)


# speedup vs baseline: 1.2595x; 1.2595x over previous
import functools

import jax
import jax.numpy as jnp
from jax import lax
from jax.experimental import pallas as pl
from jax.experimental.pallas import tpu as pltpu

BATCH = 16384
HIDDEN = 64
GAMMA = 12.0

BLK = 512
NBLK = BATCH // BLK


def _body(idx_h, idx_r, idx_t, ent_hbm, rel_hbm, out_ref,
          buf_h, buf_r, buf_t, sems):
    k = pl.program_id(0)

    def issue_block(blk, par):
        base = blk * BLK

        def enqueue(r, carry):
            ih = idx_h[base + r]
            ir = idx_r[base + r]
            it = idx_t[base + r]
            pltpu.async_copy(ent_hbm.at[ih], buf_h.at[par, r], sems.at[par])
            pltpu.async_copy(rel_hbm.at[ir], buf_r.at[par, r], sems.at[par])
            pltpu.async_copy(ent_hbm.at[it], buf_t.at[par, r], sems.at[par])
            return carry

        lax.fori_loop(0, BLK, enqueue, 0, unroll=8)

    par = lax.rem(k, 2)
    nxt = lax.rem(k + 1, 2)

    @pl.when(k == 0)
    def _():
        issue_block(0, 0)

    @pl.when(k + 1 < NBLK)
    def _():
        issue_block(k + 1, nxt)

    for buf in (buf_h, buf_r, buf_t):
        pltpu.make_async_copy(
            ent_hbm.at[pl.ds(0, BLK)], buf.at[par], sems.at[par]).wait()

    h = buf_h[par]
    r = buf_r[par]
    t = buf_t[par]
    d = jnp.abs(h + r - t)
    out_ref[...] = GAMMA - jnp.sum(d, axis=1, keepdims=True)


@jax.jit
def _score(heads, rels, tails, entity_embedding, relation_embedding):
    grid_spec = pltpu.PrefetchScalarGridSpec(
        num_scalar_prefetch=3,
        grid=(NBLK,),
        in_specs=[
            pl.BlockSpec(memory_space=pl.ANY),
            pl.BlockSpec(memory_space=pl.ANY),
        ],
        out_specs=pl.BlockSpec((BLK, 1), lambda k, *prefetch: (k, 0)),
        scratch_shapes=[
            pltpu.VMEM((2, BLK, HIDDEN), jnp.float32),
            pltpu.VMEM((2, BLK, HIDDEN), jnp.float32),
            pltpu.VMEM((2, BLK, HIDDEN), jnp.float32),
            pltpu.SemaphoreType.DMA((2,)),
        ],
    )
    fn = pl.pallas_call(
        _body,
        grid_spec=grid_spec,
        out_shape=jax.ShapeDtypeStruct((BATCH, 1), jnp.float32),
        compiler_params=pltpu.CompilerParams(
            dimension_semantics=("arbitrary",)),
    )
    return fn(heads, rels, tails, entity_embedding, relation_embedding)


def kernel(sample, entity_embedding, relation_embedding):
    sample = sample.astype(jnp.int32)
    heads = sample[:, 0]
    rels = sample[:, 1]
    tails = sample[:, 2]
    return _score(heads, rels, tails, entity_embedding, relation_embedding)
